# baseline (device time: 166955 ns/iter reference)
import jax
import jax.numpy as jnp
from jax import lax
from jax.experimental import pallas as pl
from jax.experimental.pallas import tpu as pltpu

N_DEV = 16
B, SQ, SKV = 2, 512, 512
H_PER = 8
DH = 64
D_MODEL = 768
HD = H_PER * DH
CHUNK = B * SQ // N_DEV
CPB = SQ // CHUNK
WINDOW = 128
HOPS = N_DEV - 1


def kernel(x, Wq, K_ext, V_ext, Wo):
    i = lax.axis_index("i")
    K = lax.dynamic_slice_in_dim(K_ext, i * H_PER, H_PER, axis=2)
    V = lax.dynamic_slice_in_dim(V_ext, i * H_PER, H_PER, axis=2)
    K = jnp.swapaxes(K.astype(jnp.bfloat16), 2, 3).reshape(B, SKV, HD)
    V = jnp.swapaxes(V.astype(jnp.bfloat16), 2, 3).reshape(B, SKV, HD)
    x16 = x.astype(jnp.bfloat16)
    Wq16 = Wq.astype(jnp.bfloat16)
    Wo16 = Wo.astype(jnp.bfloat16)

    def body(x_ref, wq_ref, k_ref, v_ref, wo_ref, out_ref,
             snd, rs_recv, ag_recv,
             rs_send_sems, rs_recv_sems, ag_send_sems, ag_recv_sems):
        my = lax.axis_index("i")
        left = lax.rem(my + N_DEV - 1, N_DEV)
        right = lax.rem(my + 1, N_DEV)

        barrier = pltpu.get_barrier_semaphore()
        for nbr in (left, right):
            pl.semaphore_signal(
                barrier, inc=1,
                device_id=(nbr,), device_id_type=pl.DeviceIdType.MESH,
            )
        pl.semaphore_wait(barrier, 2)

        qi = lax.broadcasted_iota(jnp.int32, (SQ, SKV), 0)
        ki = lax.broadcasted_iota(jnp.int32, (SQ, SKV), 1)
        mask = jnp.abs(qi - ki) <= WINDOW
        ci = lax.broadcasted_iota(jnp.int32, (HD, DH), 0)
        di = lax.broadcasted_iota(jnp.int32, (HD, DH), 1)
        sel = [(ci == di * H_PER + h).astype(jnp.bfloat16)
               for h in range(H_PER)]
        for b in range(B):
            q_b = jnp.dot(x_ref[b], wq_ref[:, :],
                          preferred_element_type=jnp.float32)
            q_b = q_b.astype(jnp.bfloat16)
            ctx_cols = []
            for h in range(H_PER):
                qh = q_b[:, h * DH:(h + 1) * DH]
                kh = jnp.dot(k_ref[b], sel[h],
                             preferred_element_type=jnp.float32
                             ).astype(jnp.bfloat16)
                vh = jnp.dot(v_ref[b], sel[h],
                             preferred_element_type=jnp.float32
                             ).astype(jnp.bfloat16)
                s = lax.dot_general(
                    qh, kh, (((1,), (1,)), ((), ())),
                    preferred_element_type=jnp.float32) * 0.125
                s = jnp.where(mask, s, -1e9)
                s = s - jnp.max(s, axis=-1, keepdims=True)
                w = jnp.exp(s)
                w = (w / jnp.sum(w, axis=-1, keepdims=True)).astype(jnp.bfloat16)
                ctx_cols.append(jnp.dot(w, vh,
                                        preferred_element_type=jnp.float32))
            ctx_b = jnp.concatenate(ctx_cols, axis=1).astype(jnp.bfloat16)
            out_ref[b, :, :] = jnp.dot(
                ctx_b, wo_ref[:, :], preferred_element_type=jnp.float32)

        def chunk(c):
            return c // CPB, pl.ds(lax.rem(c, CPB) * CHUNK, CHUNK)

        def send(src, dst, ssem, rsem, dev):
            rdma = pltpu.make_async_remote_copy(
                src_ref=src, dst_ref=dst, send_sem=ssem, recv_sem=rsem,
                device_id=(dev,), device_id_type=pl.DeviceIdType.MESH)
            rdma.start()
            return rdma

        rs_rdmas = []
        for s in range(HOPS):
            bc, rc = chunk(lax.rem(my - s + 2 * N_DEV, N_DEV))
            snd[s] = out_ref[bc, rc, :].astype(jnp.bfloat16)
            rd = send(snd.at[s], rs_recv.at[s],
                      rs_send_sems.at[s], rs_recv_sems.at[s], right)
            rs_rdmas.append(rd)
            rd.wait_recv()
            bc, rc = chunk(lax.rem(my - s - 1 + 2 * N_DEV, N_DEV))
            out_ref[bc, rc, :] = (out_ref[bc, rc, :]
                                  + rs_recv[s].astype(jnp.float32))
        for rd in rs_rdmas:
            rd.wait_send()

        ag_rdmas = []
        for s in range(HOPS):
            if s == 0:
                bc, rc = chunk(lax.rem(my + 1, N_DEV))
                snd[HOPS] = out_ref[bc, rc, :].astype(jnp.bfloat16)
                src = snd.at[HOPS]
            else:
                src = ag_recv.at[s - 1]
            rd = send(src, ag_recv.at[s],
                      ag_send_sems.at[s], ag_recv_sems.at[s], right)
            ag_rdmas.append(rd)
            rd.wait_recv()
        for s in range(HOPS):
            bc, rc = chunk(lax.rem(my - s + 2 * N_DEV, N_DEV))
            out_ref[bc, rc, :] = ag_recv[s].astype(jnp.float32)
        for rd in ag_rdmas:
            rd.wait_send()

    vmem = pl.BlockSpec(memory_space=pltpu.MemorySpace.VMEM)
    return pl.pallas_call(
        body,
        out_shape=jax.ShapeDtypeStruct((B, SQ, D_MODEL), jnp.float32),
        in_specs=[vmem] * 5,
        out_specs=vmem,
        scratch_shapes=[
            pltpu.VMEM((HOPS + 1, CHUNK, D_MODEL), jnp.bfloat16),
            pltpu.VMEM((HOPS, CHUNK, D_MODEL), jnp.bfloat16),
            pltpu.VMEM((HOPS, CHUNK, D_MODEL), jnp.bfloat16),
        ] + [pltpu.SemaphoreType.DMA((HOPS,))] * 4,
        compiler_params=pltpu.CompilerParams(collective_id=0),
    )(x16, Wq16, K, V, Wo16)


# device time: 160706 ns/iter; 1.0389x vs baseline; 1.0389x over previous
import jax
import jax.numpy as jnp
from jax import lax
from jax.experimental import pallas as pl
from jax.experimental.pallas import tpu as pltpu

N_DEV = 16
B, SQ, SKV = 2, 512, 512
H_PER = 8
DH = 64
D_MODEL = 768
HD = H_PER * DH
CHUNK = B * SQ // N_DEV
CPB = SQ // CHUNK
WINDOW = 128
HOPS = N_DEV - 1


def kernel(x, Wq, K_ext, V_ext, Wo):
    i = lax.axis_index("i")
    K = lax.dynamic_slice_in_dim(K_ext, i * H_PER, H_PER, axis=2)
    V = lax.dynamic_slice_in_dim(V_ext, i * H_PER, H_PER, axis=2)
    K = K.astype(jnp.bfloat16)
    V = V.astype(jnp.bfloat16)
    x16 = x.astype(jnp.bfloat16)
    Wq16 = Wq.astype(jnp.bfloat16)
    Wo16 = Wo.astype(jnp.bfloat16)

    def body(x_ref, wq_ref, k_ref, v_ref, wo_ref, out_ref,
             snd, rs_recv, ag_recv,
             rs_send_sems, rs_recv_sems, ag_send_sems, ag_recv_sems):
        my = lax.axis_index("i")
        left = lax.rem(my + N_DEV - 1, N_DEV)
        right = lax.rem(my + 1, N_DEV)

        barrier = pltpu.get_barrier_semaphore()
        for nbr in (left, right):
            pl.semaphore_signal(
                barrier, inc=1,
                device_id=(nbr,), device_id_type=pl.DeviceIdType.MESH,
            )
        pl.semaphore_wait(barrier, 2)

        qi = lax.broadcasted_iota(jnp.int32, (SQ, SKV), 0)
        ki = lax.broadcasted_iota(jnp.int32, (SQ, SKV), 1)
        mask = jnp.abs(qi - ki) <= WINDOW
        for b in range(B):
            q_b = jnp.dot(x_ref[b], wq_ref[:, :],
                          preferred_element_type=jnp.float32)
            q_b = q_b.astype(jnp.bfloat16)
            ctx_cols = []
            for h in range(H_PER):
                qh = q_b[:, h * DH:(h + 1) * DH]
                kh = k_ref[b, :, h, :]
                vh = v_ref[b, :, h, :]
                s = lax.dot_general(
                    qh, kh, (((1,), (1,)), ((), ())),
                    preferred_element_type=jnp.float32) * 0.125
                s = jnp.where(mask, s, -1e9)
                s = s - jnp.max(s, axis=-1, keepdims=True)
                w = jnp.exp(s)
                w = (w / jnp.sum(w, axis=-1, keepdims=True)).astype(jnp.bfloat16)
                ctx_cols.append(jnp.dot(w, vh,
                                        preferred_element_type=jnp.float32))
            ctx_b = jnp.concatenate(ctx_cols, axis=1).astype(jnp.bfloat16)
            out_ref[b, :, :] = jnp.dot(
                ctx_b, wo_ref[:, :], preferred_element_type=jnp.float32)

        def chunk(c):
            return c // CPB, pl.ds(lax.rem(c, CPB) * CHUNK, CHUNK)

        def send(src, dst, ssem, rsem, dev):
            rdma = pltpu.make_async_remote_copy(
                src_ref=src, dst_ref=dst, send_sem=ssem, recv_sem=rsem,
                device_id=(dev,), device_id_type=pl.DeviceIdType.MESH)
            rdma.start()
            return rdma

        rs_rdmas = []
        for s in range(HOPS):
            bc, rc = chunk(lax.rem(my - s + 2 * N_DEV, N_DEV))
            snd[s] = out_ref[bc, rc, :].astype(jnp.bfloat16)
            rd = send(snd.at[s], rs_recv.at[s],
                      rs_send_sems.at[s], rs_recv_sems.at[s], right)
            rs_rdmas.append(rd)
            rd.wait_recv()
            bc, rc = chunk(lax.rem(my - s - 1 + 2 * N_DEV, N_DEV))
            out_ref[bc, rc, :] = (out_ref[bc, rc, :]
                                  + rs_recv[s].astype(jnp.float32))
        for rd in rs_rdmas:
            rd.wait_send()

        ag_rdmas = []
        for s in range(HOPS):
            if s == 0:
                bc, rc = chunk(lax.rem(my + 1, N_DEV))
                snd[HOPS] = out_ref[bc, rc, :].astype(jnp.bfloat16)
                src = snd.at[HOPS]
            else:
                src = ag_recv.at[s - 1]
            rd = send(src, ag_recv.at[s],
                      ag_send_sems.at[s], ag_recv_sems.at[s], right)
            ag_rdmas.append(rd)
            rd.wait_recv()
        for s in range(HOPS):
            bc, rc = chunk(lax.rem(my - s + 2 * N_DEV, N_DEV))
            out_ref[bc, rc, :] = ag_recv[s].astype(jnp.float32)
        for rd in ag_rdmas:
            rd.wait_send()

    vmem = pl.BlockSpec(memory_space=pltpu.MemorySpace.VMEM)
    return pl.pallas_call(
        body,
        out_shape=jax.ShapeDtypeStruct((B, SQ, D_MODEL), jnp.float32),
        in_specs=[vmem] * 5,
        out_specs=vmem,
        scratch_shapes=[
            pltpu.VMEM((HOPS + 1, CHUNK, D_MODEL), jnp.bfloat16),
            pltpu.VMEM((HOPS, CHUNK, D_MODEL), jnp.bfloat16),
            pltpu.VMEM((HOPS, CHUNK, D_MODEL), jnp.bfloat16),
        ] + [pltpu.SemaphoreType.DMA((HOPS,))] * 4,
        compiler_params=pltpu.CompilerParams(collective_id=0),
    )(x16, Wq16, K, V, Wo16)


# device time: 112928 ns/iter; 1.4784x vs baseline; 1.4231x over previous
import jax
import jax.numpy as jnp
from jax import lax
from jax.experimental import pallas as pl
from jax.experimental.pallas import tpu as pltpu

N_DEV = 16
B, SQ, SKV = 2, 512, 512
H_PER = 8
DH = 64
D_MODEL = 768
HALF = D_MODEL // 2
QROWS = B * SQ // 4
WINDOW = 128
F32 = jnp.float32
BF16 = jnp.bfloat16


def kernel(x, Wq, K_ext, V_ext, Wo):
    i = lax.axis_index("i")
    K = lax.dynamic_slice_in_dim(K_ext, i * H_PER, H_PER, axis=2)
    V = lax.dynamic_slice_in_dim(V_ext, i * H_PER, H_PER, axis=2)
    K = K.astype(BF16)
    V = V.astype(BF16)
    x16 = x.astype(BF16)
    Wq16 = Wq.astype(BF16)
    Wo16 = Wo.astype(BF16)

    def body(x_ref, wq_ref, k_ref, v_ref, wo_ref, out_ref,
             p1_snd_r, p1_snd_l, p1_rcv_r, p1_rcv_l,
             p2_snd_r, p2_snd_l, p2_rcv_r, p2_rcv_l,
             p3_snd_r, p3_snd_l, p3_rcv_r, p3_rcv_l,
             p1_ss_r, p1_rs_r, p1_ss_l, p1_rs_l,
             p2_ss_r, p2_rs_r, p2_ss_l, p2_rs_l,
             p3_ss_r, p3_rs_r, p3_ss_l, p3_rs_l):
        my = lax.axis_index("i")
        q = lax.rem(my, 4)
        z = my // 4
        zb0 = lax.rem(z, 2)
        zb1 = z // 2
        pnext = z * 4 + lax.rem(q + 1, 4)
        pprev = z * 4 + lax.rem(q + 3, 4)
        zp1 = (z + 1 - 2 * zb0) * 4 + q
        zp2 = (z + 2 - 4 * zb1) * 4 + q

        barrier = pltpu.get_barrier_semaphore()
        for nbr in (pnext, pprev, zp1, zp2):
            pl.semaphore_signal(
                barrier, inc=1,
                device_id=(nbr,), device_id_type=pl.DeviceIdType.MESH,
            )
        pl.semaphore_wait(barrier, 4)

        qi = lax.broadcasted_iota(jnp.int32, (SQ, SKV), 0)
        ki = lax.broadcasted_iota(jnp.int32, (SQ, SKV), 1)
        mask = jnp.abs(qi - ki) <= WINDOW
        for b in range(B):
            q_b = jnp.dot(x_ref[b], wq_ref[:, :],
                          preferred_element_type=F32)
            q_b = q_b.astype(BF16)
            ctx_cols = []
            for h in range(H_PER):
                qh = q_b[:, h * DH:(h + 1) * DH]
                kh = k_ref[b, :, h, :]
                vh = v_ref[b, :, h, :]
                s = lax.dot_general(
                    qh, kh, (((1,), (1,)), ((), ())),
                    preferred_element_type=F32) * 0.125
                s = jnp.where(mask, s, -1e9)
                s = s - jnp.max(s, axis=-1, keepdims=True)
                w = jnp.exp(s)
                w = (w / jnp.sum(w, axis=-1, keepdims=True)).astype(BF16)
                ctx_cols.append(jnp.dot(w, vh, preferred_element_type=F32))
            ctx_b = jnp.concatenate(ctx_cols, axis=1).astype(BF16)
            out_ref[b, :, :] = jnp.dot(
                ctx_b, wo_ref[:, :], preferred_element_type=F32)

        R = slice(0, HALF)
        L = slice(HALF, D_MODEL)

        def qtr(r, off, n):
            return r // 2, pl.ds(lax.rem(r, 2) * QROWS + off, n)

        def send(src, dst, ssem, rsem, dev):
            rdma = pltpu.make_async_remote_copy(
                src_ref=src, dst_ref=dst, send_sem=ssem, recv_sem=rsem,
                device_id=(dev,), device_id_type=pl.DeviceIdType.MESH)
            rdma.start()
            return rdma

        p1 = []
        for s in range(3):
            bR, rR_ = qtr(lax.rem(q - s + 8, 4), 0, QROWS)
            bL, rL_ = qtr(lax.rem(q + s, 4), 0, QROWS)
            p1_snd_r[s] = out_ref[bR, rR_, R].astype(BF16)
            p1_snd_l[s] = out_ref[bL, rL_, L].astype(BF16)
            rd_r = send(p1_snd_r.at[s], p1_rcv_r.at[s],
                        p1_ss_r.at[s], p1_rs_r.at[s], pnext)
            rd_l = send(p1_snd_l.at[s], p1_rcv_l.at[s],
                        p1_ss_l.at[s], p1_rs_l.at[s], pprev)
            p1 += [rd_r, rd_l]
            rd_r.wait_recv()
            rd_l.wait_recv()
            bR, rR_ = qtr(lax.rem(q - s - 1 + 8, 4), 0, QROWS)
            bL, rL_ = qtr(lax.rem(q + s + 1, 4), 0, QROWS)
            out_ref[bR, rR_, R] = out_ref[bR, rR_, R] + p1_rcv_r[s].astype(F32)
            out_ref[bL, rL_, L] = out_ref[bL, rL_, L] + p1_rcv_l[s].astype(F32)
        for rd in p1:
            rd.wait_send()
        rqR = lax.rem(q + 1, 4)
        rqL = lax.rem(q + 3, 4)

        keep = zb0 * 128
        soff = 128 - keep
        own = keep + zb1 * 64
        poff = keep + 64 - zb1 * 64
        other = 128 - keep
        p2 = []
        pieces = ((rqR, R, p2_snd_r, p2_rcv_r, p2_ss_r, p2_rs_r),
                  (rqL, L, p2_snd_l, p2_rcv_l, p2_ss_l, p2_rs_l))
        steps = ((0, soff, 128, zp1), (1, keep + 64 - zb1 * 64, 64, zp2),
                 (2, own, 64, zp2), (3, keep, 128, zp1))
        dests = ((0, keep, 128), (1, keep + zb1 * 64, 64),
                 (2, poff, 64), (3, other, 128))
        for (s, so, n, dev), (_, do, dn) in zip(steps, dests):
            rds = []
            for (r, C, snd, rcv, ss, rs) in pieces:
                bq, rr = qtr(r, so, n)
                snd[s, pl.ds(0, n)] = out_ref[bq, rr, C].astype(BF16)
                rds.append(send(snd.at[s, pl.ds(0, n)],
                                rcv.at[s, pl.ds(0, n)],
                                ss.at[s], rs.at[s], dev))
            p2 += rds
            for rd in rds:
                rd.wait_recv()
            for (r, C, snd, rcv, ss, rs) in pieces:
                bq, rr = qtr(r, do, dn)
                got = rcv[s, pl.ds(0, dn)].astype(F32)
                if s < 2:
                    out_ref[bq, rr, C] = out_ref[bq, rr, C] + got
                else:
                    out_ref[bq, rr, C] = got
        for rd in p2:
            rd.wait_send()

        p3 = []
        for s in range(3):
            if s == 0:
                bR, rR_ = qtr(rqR, 0, QROWS)
                bL, rL_ = qtr(rqL, 0, QROWS)
                p3_snd_r[0] = out_ref[bR, rR_, R].astype(BF16)
                p3_snd_l[0] = out_ref[bL, rL_, L].astype(BF16)
                src_r, src_l = p3_snd_r.at[0], p3_snd_l.at[0]
            else:
                src_r, src_l = p3_rcv_r.at[s - 1], p3_rcv_l.at[s - 1]
            rd_r = send(src_r, p3_rcv_r.at[s],
                        p3_ss_r.at[s], p3_rs_r.at[s], pnext)
            rd_l = send(src_l, p3_rcv_l.at[s],
                        p3_ss_l.at[s], p3_rs_l.at[s], pprev)
            p3 += [rd_r, rd_l]
            rd_r.wait_recv()
            rd_l.wait_recv()
        for s in range(3):
            bR, rR_ = qtr(lax.rem(q - s + 8, 4), 0, QROWS)
            bL, rL_ = qtr(lax.rem(q + s, 4), 0, QROWS)
            out_ref[bR, rR_, R] = p3_rcv_r[s].astype(F32)
            out_ref[bL, rL_, L] = p3_rcv_l[s].astype(F32)
        for rd in p3:
            rd.wait_send()

    vmem = pl.BlockSpec(memory_space=pltpu.MemorySpace.VMEM)
    return pl.pallas_call(
        body,
        out_shape=jax.ShapeDtypeStruct((B, SQ, D_MODEL), F32),
        in_specs=[vmem] * 5,
        out_specs=vmem,
        scratch_shapes=[
            pltpu.VMEM((3, QROWS, HALF), BF16),
            pltpu.VMEM((3, QROWS, HALF), BF16),
            pltpu.VMEM((3, QROWS, HALF), BF16),
            pltpu.VMEM((3, QROWS, HALF), BF16),
            pltpu.VMEM((4, 128, HALF), BF16),
            pltpu.VMEM((4, 128, HALF), BF16),
            pltpu.VMEM((4, 128, HALF), BF16),
            pltpu.VMEM((4, 128, HALF), BF16),
            pltpu.VMEM((1, QROWS, HALF), BF16),
            pltpu.VMEM((1, QROWS, HALF), BF16),
            pltpu.VMEM((3, QROWS, HALF), BF16),
            pltpu.VMEM((3, QROWS, HALF), BF16),
            pltpu.SemaphoreType.DMA((3,)), pltpu.SemaphoreType.DMA((3,)),
            pltpu.SemaphoreType.DMA((3,)), pltpu.SemaphoreType.DMA((3,)),
            pltpu.SemaphoreType.DMA((4,)), pltpu.SemaphoreType.DMA((4,)),
            pltpu.SemaphoreType.DMA((4,)), pltpu.SemaphoreType.DMA((4,)),
            pltpu.SemaphoreType.DMA((3,)), pltpu.SemaphoreType.DMA((3,)),
            pltpu.SemaphoreType.DMA((3,)), pltpu.SemaphoreType.DMA((3,)),
        ],
        compiler_params=pltpu.CompilerParams(collective_id=0),
    )(x16, Wq16, K, V, Wo16)
